# baseline (speedup 1.0000x reference)
import functools

import jax
import jax.numpy as jnp
from jax import lax
from jax.experimental import pallas as pl
from jax.experimental.pallas import tpu as pltpu
from jax.experimental.pallas import tpu_sc as plsc

NN = 10000
NP = 10240
D0 = 128
D1 = 16
D2 = 32
E = 320000
NW = 32
EB = 256
CH0 = 44
CH1 = 36
TOTC = 16 * (CH0 + CH1)
EP = TOTC * EB
RPS = NP // 16
NBUF = 4
DCH0 = 48
DCH1 = 32


_mesh = plsc.VectorSubcoreMesh(core_axis_name="c", subcore_axis_name="s")
_f32 = jnp.float32
_sc_params = pltpu.CompilerParams(use_tc_tiling_on_sc=False)


@functools.partial(
    pl.kernel,
    out_type=jax.ShapeDtypeStruct((2, NP, D1), _f32),
    mesh=_mesh,
    scratch_types=[
        pltpu.VMEM((DCH0, EB), jnp.int32),
        pltpu.VMEM((EB, D1), _f32),
        pltpu.VMEM((RPS, D1), _f32),
        pltpu.VMEM_SHARED((NP, D1), _f32),
        pltpu.SemaphoreType.DMA,
    ],
    compiler_params=_sc_params,
)
def _deg_pass(dst_hbm, out_hbm, dst_v, ones_v, buf_v, acc_sh, sem):
    c = lax.axis_index("c")
    s = lax.axis_index("s")

    @pl.loop(0, EB)
    def _(i):
        ones_v[i, :] = jnp.ones((D1,), _f32)

    @pl.loop(0, RPS)
    def _(i):
        buf_v[i, :] = jnp.zeros((D1,), _f32)

    pltpu.sync_copy(buf_v, acc_sh.at[pl.ds(s * RPS, RPS)])

    def run(base, ch):
        pltpu.sync_copy(dst_hbm.at[pl.ds(base, ch)], dst_v.at[pl.ds(0, ch)])
        plsc.subcore_barrier()

        @pl.loop(0, ch, step=4)
        def _(g):
            for b in range(4):
                pltpu.async_copy(ones_v, acc_sh.at[dst_v.at[g + b]], sem,
                                 add=True)
            for b in range(4):
                pltpu.make_async_copy(ones_v, acc_sh.at[dst_v.at[g + b]],
                                      sem).wait()

    @pl.when(c == 0)
    def _():
        run(s * DCH0, DCH0)

    @pl.when(c == 1)
    def _():
        run(16 * DCH0 + s * DCH1, DCH1)

    plsc.subcore_barrier()
    pltpu.sync_copy(acc_sh.at[pl.ds(s * RPS, RPS)], buf_v)
    pltpu.sync_copy(buf_v, out_hbm.at[c, pl.ds(s * RPS, RPS)])


@functools.partial(
    pl.kernel,
    out_type=jax.ShapeDtypeStruct((2, NP, D1), _f32),
    mesh=_mesh,
    scratch_types=[
        pltpu.VMEM((CH0, EB), jnp.int32),
        pltpu.VMEM((CH0, EB), jnp.int32),
        pltpu.VMEM((NBUF, EB, D1), _f32),
        pltpu.VMEM((RPS, D1), _f32),
        pltpu.VMEM_SHARED((NP, D1), _f32),
        pltpu.VMEM_SHARED((NP, D1), _f32),
        pltpu.SemaphoreType.DMA((NBUF,)),
        pltpu.SemaphoreType.DMA((NBUF,)),
    ],
    compiler_params=_sc_params,
)
def _seg_sum(y_hbm, src_hbm, dst_hbm, out_hbm, src_v, dst_v, rows_v, buf_v,
             acc_sh, y_sh, gsem, ssem):
    c = lax.axis_index("c")
    s = lax.axis_index("s")

    pltpu.sync_copy(y_hbm.at[pl.ds(s * RPS, RPS)], buf_v)
    pltpu.sync_copy(buf_v, y_sh.at[pl.ds(s * RPS, RPS)])

    @pl.loop(0, RPS)
    def _(i):
        buf_v[i, :] = jnp.zeros((D1,), _f32)

    pltpu.sync_copy(buf_v, acc_sh.at[pl.ds(s * RPS, RPS)])

    def run(base, ch):
        pltpu.sync_copy(src_hbm.at[pl.ds(base, ch)], src_v.at[pl.ds(0, ch)])
        pltpu.sync_copy(dst_hbm.at[pl.ds(base, ch)], dst_v.at[pl.ds(0, ch)])
        plsc.subcore_barrier()

        for b in range(NBUF):
            pltpu.async_copy(y_sh.at[src_v.at[b]], rows_v.at[b], gsem.at[b])

        @pl.loop(0, ch, step=NBUF)
        def _(g):
            descs = []
            for b in range(NBUF):
                j = g + b
                pltpu.make_async_copy(
                    y_sh.at[src_v.at[j]], rows_v.at[b], gsem.at[b]).wait()
                descs.append(pltpu.async_copy(
                    rows_v.at[b], acc_sh.at[dst_v.at[j]], ssem.at[b],
                    add=True))
            for b in range(NBUF):
                nj = g + NBUF + b

                @pl.when(nj < ch)
                def _(b=b, nj=nj):
                    descs[b].wait()
                    pltpu.async_copy(y_sh.at[src_v.at[nj]], rows_v.at[b],
                                     gsem.at[b])

        for b in range(NBUF):
            j = ch - NBUF + b
            pltpu.make_async_copy(
                rows_v.at[b], acc_sh.at[dst_v.at[j]], ssem.at[b]).wait()

    @pl.when(c == 0)
    def _():
        run(s * CH0, CH0)

    @pl.when(c == 1)
    def _():
        run(16 * CH0 + s * CH1, CH1)

    plsc.subcore_barrier()
    pltpu.sync_copy(acc_sh.at[pl.ds(s * RPS, RPS)], buf_v)
    pltpu.sync_copy(buf_v, out_hbm.at[c, pl.ds(s * RPS, RPS)])


V = NP * D1 // 128


def _tc_xw_body(x_ref, w1_ref, xw_ref):
    xw = jnp.dot(x_ref[...], w1_ref[...], preferred_element_type=_f32)
    xw_ref[...] = jnp.concatenate([xw, jnp.zeros((NP - NN, D1), _f32)], axis=0)


_tc_xw = pl.pallas_call(
    _tc_xw_body,
    out_shape=jax.ShapeDtypeStruct((NP, D1), _f32),
)


def _tc_scale_body(degp_ref, xw_ref, y_ref, dinv_ref):
    deg = degp_ref[0] + degp_ref[1] + 1.0
    dinv = lax.rsqrt(deg)
    y_ref[...] = xw_ref[...] * dinv
    dinv_ref[...] = dinv


_tc_scale = pl.pallas_call(
    _tc_scale_body,
    out_shape=[jax.ShapeDtypeStruct((V, 128), _f32),
               jax.ShapeDtypeStruct((V, 128), _f32)],
)


def _tc2_body(sp_ref, y_ref, dinv_ref, b1_ref, z_ref):
    agg = dinv_ref[...] * (sp_ref[0] + sp_ref[1] + y_ref[...])
    h = jnp.maximum(agg + b1_ref[...], 0.0)
    z_ref[...] = dinv_ref[...] * h


_tc2 = pl.pallas_call(
    _tc2_body,
    out_shape=jax.ShapeDtypeStruct((V, 128), _f32),
)


def _tc3_body(tp_ref, z_ref, dinv_ref, w2b_ref, b2b_ref, o_ref):
    agg = dinv_ref[...] * (tp_ref[0] + tp_ref[1] + z_ref[...])
    for i in range(128 // D1):
        blk = jnp.dot(agg[:, D1 * i:D1 * (i + 1)], w2b_ref[...],
                      preferred_element_type=_f32) + b2b_ref[...]
        m = jnp.max(blk, axis=1, keepdims=True)
        lse = jnp.log(jnp.sum(jnp.exp(blk - m), axis=1, keepdims=True)) + m
        o_ref[:, D2 * i:D2 * (i + 1)] = blk - lse


_tc3 = pl.pallas_call(
    _tc3_body,
    out_shape=jax.ShapeDtypeStruct((V, 2 * 128), _f32),
)


def kernel(x, edge_index, W1, b1, W2, b2):
    ei = edge_index.astype(jnp.int32)
    pad = jnp.full((EP - E,), NN, jnp.int32)
    src = jnp.concatenate([ei[0], pad]).reshape(TOTC, EB)
    dst = jnp.concatenate([ei[1], pad]).reshape(TOTC, EB)
    b1v = jnp.tile(b1, 128 // D1).reshape(1, 128)

    degp = _deg_pass(dst)
    degpv = degp.reshape(2, V, 128)
    xwv = _tc_xw(x, W1).reshape(V, 128)
    yv, dinvv = _tc_scale(degpv, xwv)
    sp = _seg_sum(yv.reshape(NP, D1), src, dst)
    zv = _tc2(sp.reshape(2, V, 128), yv, dinvv, b1v)
    tp = _seg_sum(zv.reshape(NP, D1), src, dst)
    out = _tc3(tp.reshape(2, V, 128), zv, dinvv, W2, b2.reshape(1, D2))
    return out.reshape(NP, D2)[:NN]


# speedup vs baseline: 1.0792x; 1.0792x over previous
import functools

import jax
import jax.numpy as jnp
from jax import lax
from jax.experimental import pallas as pl
from jax.experimental.pallas import tpu as pltpu
from jax.experimental.pallas import tpu_sc as plsc

NN = 10000
NP = 10240
D0 = 128
D1 = 16
D2 = 32
E = 320000
NW = 32
EB = 320
CH0 = 36
CH1 = 28
TOTC = 16 * (CH0 + CH1)
EP = TOTC * EB
RPS = NP // 16
NBUF = 4
DCH0 = 40
DCH1 = 24


_mesh = plsc.VectorSubcoreMesh(core_axis_name="c", subcore_axis_name="s")
_f32 = jnp.float32
_sc_params = pltpu.CompilerParams(use_tc_tiling_on_sc=False)


@functools.partial(
    pl.kernel,
    out_type=jax.ShapeDtypeStruct((2, NP, D1), _f32),
    mesh=_mesh,
    scratch_types=[
        pltpu.VMEM((DCH0, EB), jnp.int32),
        pltpu.VMEM((EB, D1), _f32),
        pltpu.VMEM((RPS, D1), _f32),
        pltpu.VMEM_SHARED((NP, D1), _f32),
        pltpu.SemaphoreType.DMA,
    ],
    compiler_params=_sc_params,
)
def _deg_pass(e_hbm, out_hbm, dst_v, ones_v, buf_v, acc_sh, sem):
    c = lax.axis_index("c")
    s = lax.axis_index("s")

    @pl.loop(0, EB)
    def _(i):
        ones_v[i, :] = jnp.ones((D1,), _f32)

    @pl.loop(0, RPS)
    def _(i):
        buf_v[i, :] = jnp.zeros((D1,), _f32)

    pltpu.sync_copy(buf_v, acc_sh.at[pl.ds(s * RPS, RPS)])

    def run(base, ch):
        pltpu.sync_copy(e_hbm.at[1, pl.ds(base, ch)], dst_v.at[pl.ds(0, ch)])
        plsc.subcore_barrier()

        @pl.loop(0, ch, step=4)
        def _(g):
            for b in range(4):
                pltpu.async_copy(ones_v, acc_sh.at[dst_v.at[g + b]], sem,
                                 add=True)
            for b in range(4):
                pltpu.make_async_copy(ones_v, acc_sh.at[dst_v.at[g + b]],
                                      sem).wait()

    @pl.when(c == 0)
    def _():
        run(s * DCH0, DCH0)

    @pl.when(c == 1)
    def _():
        run(16 * DCH0 + s * DCH1, DCH1)

    plsc.subcore_barrier()
    pltpu.sync_copy(acc_sh.at[pl.ds(s * RPS, RPS)], buf_v)
    pltpu.sync_copy(buf_v, out_hbm.at[c, pl.ds(s * RPS, RPS)])


@functools.partial(
    pl.kernel,
    out_type=jax.ShapeDtypeStruct((2, NP, D1), _f32),
    mesh=_mesh,
    scratch_types=[
        pltpu.VMEM((CH0, EB), jnp.int32),
        pltpu.VMEM((CH0, EB), jnp.int32),
        pltpu.VMEM((NBUF, EB, D1), _f32),
        pltpu.VMEM((RPS, D1), _f32),
        pltpu.VMEM_SHARED((NP, D1), _f32),
        pltpu.VMEM_SHARED((NP, D1), _f32),
        pltpu.SemaphoreType.DMA((NBUF,)),
        pltpu.SemaphoreType.DMA((NBUF,)),
    ],
    compiler_params=_sc_params,
)
def _seg_sum(y_hbm, e_hbm, out_hbm, src_v, dst_v, rows_v, buf_v,
             acc_sh, y_sh, gsem, ssem):
    c = lax.axis_index("c")
    s = lax.axis_index("s")

    pltpu.sync_copy(y_hbm.at[pl.ds(s * RPS, RPS)], buf_v)
    pltpu.sync_copy(buf_v, y_sh.at[pl.ds(s * RPS, RPS)])

    @pl.loop(0, RPS)
    def _(i):
        buf_v[i, :] = jnp.zeros((D1,), _f32)

    pltpu.sync_copy(buf_v, acc_sh.at[pl.ds(s * RPS, RPS)])

    def run(base, ch):
        pltpu.sync_copy(e_hbm.at[0, pl.ds(base, ch)], src_v.at[pl.ds(0, ch)])
        pltpu.sync_copy(e_hbm.at[1, pl.ds(base, ch)], dst_v.at[pl.ds(0, ch)])
        plsc.subcore_barrier()

        for b in range(NBUF):
            pltpu.async_copy(y_sh.at[src_v.at[b]], rows_v.at[b], gsem.at[b])

        @pl.loop(0, ch, step=NBUF)
        def _(g):
            descs = []
            for b in range(NBUF):
                j = g + b
                pltpu.make_async_copy(
                    y_sh.at[src_v.at[j]], rows_v.at[b], gsem.at[b]).wait()
                descs.append(pltpu.async_copy(
                    rows_v.at[b], acc_sh.at[dst_v.at[j]], ssem.at[b],
                    add=True))
            for b in range(NBUF):
                nj = g + NBUF + b

                @pl.when(nj < ch)
                def _(b=b, nj=nj):
                    descs[b].wait()
                    pltpu.async_copy(y_sh.at[src_v.at[nj]], rows_v.at[b],
                                     gsem.at[b])

        for b in range(NBUF):
            j = ch - NBUF + b
            pltpu.make_async_copy(
                rows_v.at[b], acc_sh.at[dst_v.at[j]], ssem.at[b]).wait()

    @pl.when(c == 0)
    def _():
        run(s * CH0, CH0)

    @pl.when(c == 1)
    def _():
        run(16 * CH0 + s * CH1, CH1)

    plsc.subcore_barrier()
    pltpu.sync_copy(acc_sh.at[pl.ds(s * RPS, RPS)], buf_v)
    pltpu.sync_copy(buf_v, out_hbm.at[c, pl.ds(s * RPS, RPS)])


V = NP * D1 // 128


def _tc_xw_body(x_ref, w1_ref, xw_ref):
    xw = jnp.dot(x_ref[...], w1_ref[...], preferred_element_type=_f32)
    xw_ref[...] = jnp.concatenate([xw, jnp.zeros((NP - NN, D1), _f32)], axis=0)


_tc_xw = pl.pallas_call(
    _tc_xw_body,
    out_shape=jax.ShapeDtypeStruct((NP, D1), _f32),
)


def _tc_scale_body(degp_ref, xw_ref, y_ref, dinv_ref):
    deg = degp_ref[0] + degp_ref[1] + 1.0
    dinv = lax.rsqrt(deg)
    y_ref[...] = xw_ref[...] * dinv
    dinv_ref[...] = dinv


_tc_scale = pl.pallas_call(
    _tc_scale_body,
    out_shape=[jax.ShapeDtypeStruct((V, 128), _f32),
               jax.ShapeDtypeStruct((V, 128), _f32)],
)


def _tc2_body(sp_ref, y_ref, dinv_ref, b1_ref, z_ref):
    agg = dinv_ref[...] * (sp_ref[0] + sp_ref[1] + y_ref[...])
    h = jnp.maximum(agg + b1_ref[...], 0.0)
    z_ref[...] = dinv_ref[...] * h


_tc2 = pl.pallas_call(
    _tc2_body,
    out_shape=jax.ShapeDtypeStruct((V, 128), _f32),
)


def _tc3_body(tp_ref, z_ref, dinv_ref, w2b_ref, b2b_ref, o_ref):
    agg = dinv_ref[...] * (tp_ref[0] + tp_ref[1] + z_ref[...])
    for i in range(128 // D1):
        blk = jnp.dot(agg[:, D1 * i:D1 * (i + 1)], w2b_ref[...],
                      preferred_element_type=_f32) + b2b_ref[...]
        m = jnp.max(blk, axis=1, keepdims=True)
        lse = jnp.log(jnp.sum(jnp.exp(blk - m), axis=1, keepdims=True)) + m
        o_ref[:, D2 * i:D2 * (i + 1)] = blk - lse


_tc3 = pl.pallas_call(
    _tc3_body,
    out_shape=jax.ShapeDtypeStruct((V, 2 * 128), _f32),
)


def kernel(x, edge_index, W1, b1, W2, b2):
    ei = edge_index.astype(jnp.int32)
    e3 = jnp.pad(ei, ((0, 0), (0, EP - E)),
                 constant_values=NN).reshape(2, TOTC, EB)
    b1v = jnp.tile(b1, 128 // D1).reshape(1, 128)

    degp = _deg_pass(e3)
    degpv = degp.reshape(2, V, 128)
    xwv = _tc_xw(x, W1).reshape(V, 128)
    yv, dinvv = _tc_scale(degpv, xwv)
    sp = _seg_sum(yv.reshape(NP, D1), e3)
    zv = _tc2(sp.reshape(2, V, 128), yv, dinvv, b1v)
    tp = _seg_sum(zv.reshape(NP, D1), e3)
    out = _tc3(tp.reshape(2, V, 128), zv, dinvv, W2, b2.reshape(1, D2))
    return out.reshape(NP, D2)[:NN]


# speedup vs baseline: 1.1298x; 1.0468x over previous
import functools

import jax
import jax.numpy as jnp
from jax import lax
from jax.experimental import pallas as pl
from jax.experimental.pallas import tpu as pltpu
from jax.experimental.pallas import tpu_sc as plsc

NN = 10000
NP = 10240
D0 = 128
D1 = 16
D2 = 32
E = 320000
NW = 32
EB = 320
CH0 = 36
CH1 = 28
TOTC = 16 * (CH0 + CH1)
EP = TOTC * EB
RPS = NP // 16
NBUF = 4
DCH0 = 40
DCH1 = 24


_mesh = plsc.VectorSubcoreMesh(core_axis_name="c", subcore_axis_name="s")
_f32 = jnp.float32
_sc_params = pltpu.CompilerParams(use_tc_tiling_on_sc=False)


@functools.partial(
    pl.kernel,
    out_type=jax.ShapeDtypeStruct((2, NP), _f32),
    mesh=_mesh,
    scratch_types=[
        pltpu.VMEM((DCH0, EB), jnp.int32),
        pltpu.VMEM((EB,), _f32),
        pltpu.VMEM((RPS,), _f32),
        pltpu.VMEM_SHARED((NP,), _f32),
        pltpu.SemaphoreType.DMA,
    ],
    compiler_params=_sc_params,
)
def _deg_pass(e_hbm, out_hbm, dst_v, ones_v, buf_v, acc_sh, sem):
    c = lax.axis_index("c")
    s = lax.axis_index("s")

    @pl.loop(0, EB, step=16)
    def _(i):
        ones_v[pl.ds(i, 16)] = jnp.ones((16,), _f32)

    @pl.loop(0, RPS, step=16)
    def _(i):
        buf_v[pl.ds(i, 16)] = jnp.zeros((16,), _f32)

    pltpu.sync_copy(buf_v, acc_sh.at[pl.ds(s * RPS, RPS)])

    def run(base, ch):
        pltpu.sync_copy(e_hbm.at[1, pl.ds(base, ch)], dst_v.at[pl.ds(0, ch)])
        plsc.subcore_barrier()

        @pl.loop(0, ch, step=4)
        def _(g):
            for b in range(4):
                pltpu.async_copy(ones_v, acc_sh.at[dst_v.at[g + b]], sem,
                                 add=True)
            for b in range(4):
                pltpu.make_async_copy(ones_v, acc_sh.at[dst_v.at[g + b]],
                                      sem).wait()

    @pl.when(c == 0)
    def _():
        run(s * DCH0, DCH0)

    @pl.when(c == 1)
    def _():
        run(16 * DCH0 + s * DCH1, DCH1)

    plsc.subcore_barrier()
    pltpu.sync_copy(acc_sh.at[pl.ds(s * RPS, RPS)], buf_v)
    pltpu.sync_copy(buf_v, out_hbm.at[c, pl.ds(s * RPS, RPS)])


@functools.partial(
    pl.kernel,
    out_type=jax.ShapeDtypeStruct((2, NP, D1), _f32),
    mesh=_mesh,
    scratch_types=[
        pltpu.VMEM((CH0, EB), jnp.int32),
        pltpu.VMEM((CH0, EB), jnp.int32),
        pltpu.VMEM((NBUF, EB, D1), _f32),
        pltpu.VMEM((RPS, D1), _f32),
        pltpu.VMEM_SHARED((NP, D1), _f32),
        pltpu.VMEM_SHARED((NP, D1), _f32),
        pltpu.SemaphoreType.DMA((NBUF,)),
        pltpu.SemaphoreType.DMA((NBUF,)),
    ],
    compiler_params=_sc_params,
)
def _seg_sum(y_hbm, e_hbm, out_hbm, src_v, dst_v, rows_v, buf_v,
             acc_sh, y_sh, gsem, ssem):
    c = lax.axis_index("c")
    s = lax.axis_index("s")

    pltpu.sync_copy(y_hbm.at[pl.ds(s * RPS, RPS)], buf_v)
    pltpu.sync_copy(buf_v, y_sh.at[pl.ds(s * RPS, RPS)])

    @pl.loop(0, RPS)
    def _(i):
        buf_v[i, :] = jnp.zeros((D1,), _f32)

    pltpu.sync_copy(buf_v, acc_sh.at[pl.ds(s * RPS, RPS)])

    def run(base, ch):
        pltpu.sync_copy(e_hbm.at[0, pl.ds(base, ch)], src_v.at[pl.ds(0, ch)])
        pltpu.sync_copy(e_hbm.at[1, pl.ds(base, ch)], dst_v.at[pl.ds(0, ch)])
        plsc.subcore_barrier()

        for b in range(NBUF):
            pltpu.async_copy(y_sh.at[src_v.at[b]], rows_v.at[b], gsem.at[b])

        @pl.loop(0, ch, step=NBUF)
        def _(g):
            descs = []
            for b in range(NBUF):
                j = g + b
                pltpu.make_async_copy(
                    y_sh.at[src_v.at[j]], rows_v.at[b], gsem.at[b]).wait()
                descs.append(pltpu.async_copy(
                    rows_v.at[b], acc_sh.at[dst_v.at[j]], ssem.at[b],
                    add=True))
            for b in range(NBUF):
                nj = g + NBUF + b

                @pl.when(nj < ch)
                def _(b=b, nj=nj):
                    descs[b].wait()
                    pltpu.async_copy(y_sh.at[src_v.at[nj]], rows_v.at[b],
                                     gsem.at[b])

        for b in range(NBUF):
            j = ch - NBUF + b
            pltpu.make_async_copy(
                rows_v.at[b], acc_sh.at[dst_v.at[j]], ssem.at[b]).wait()

    @pl.when(c == 0)
    def _():
        run(s * CH0, CH0)

    @pl.when(c == 1)
    def _():
        run(16 * CH0 + s * CH1, CH1)

    plsc.subcore_barrier()
    pltpu.sync_copy(acc_sh.at[pl.ds(s * RPS, RPS)], buf_v)
    pltpu.sync_copy(buf_v, out_hbm.at[c, pl.ds(s * RPS, RPS)])


V = NP * D1 // 128


def _tc_xw_body(x_ref, w1_ref, xw_ref):
    xw = jnp.dot(x_ref[...], w1_ref[...], preferred_element_type=_f32)
    xw_ref[...] = jnp.concatenate([xw, jnp.zeros((NP - NN, D1), _f32)], axis=0)


_tc_xw = pl.pallas_call(
    _tc_xw_body,
    out_shape=jax.ShapeDtypeStruct((NP, D1), _f32),
)


def _tc_scale_body(degp_ref, xw_ref, y_ref, dinv_ref):
    deg8 = degp_ref[0] + degp_ref[1]
    lane = lax.broadcasted_iota(jnp.int32, (8, 128), 1)
    row = lax.broadcasted_iota(jnp.int32, (8, 128), 0)
    rep = (lane // D1 == row).astype(_f32)
    deg = jnp.dot(deg8, rep, preferred_element_type=_f32) + 1.0
    dinv = lax.rsqrt(deg)
    y_ref[...] = xw_ref[...] * dinv
    dinv_ref[...] = dinv


_tc_scale = pl.pallas_call(
    _tc_scale_body,
    out_shape=[jax.ShapeDtypeStruct((V, 128), _f32),
               jax.ShapeDtypeStruct((V, 128), _f32)],
)


def _tc2_body(sp_ref, y_ref, dinv_ref, b1_ref, z_ref):
    agg = dinv_ref[...] * (sp_ref[0] + sp_ref[1] + y_ref[...])
    h = jnp.maximum(agg + b1_ref[...], 0.0)
    z_ref[...] = dinv_ref[...] * h


_tc2 = pl.pallas_call(
    _tc2_body,
    out_shape=jax.ShapeDtypeStruct((V, 128), _f32),
)


def _tc3_body(tp_ref, z_ref, dinv_ref, w2b_ref, b2b_ref, o_ref):
    agg = dinv_ref[...] * (tp_ref[0] + tp_ref[1] + z_ref[...])
    for i in range(128 // D1):
        blk = jnp.dot(agg[:, D1 * i:D1 * (i + 1)], w2b_ref[...],
                      preferred_element_type=_f32) + b2b_ref[...]
        m = jnp.max(blk, axis=1, keepdims=True)
        lse = jnp.log(jnp.sum(jnp.exp(blk - m), axis=1, keepdims=True)) + m
        o_ref[:, D2 * i:D2 * (i + 1)] = blk - lse


_tc3 = pl.pallas_call(
    _tc3_body,
    out_shape=jax.ShapeDtypeStruct((V, 2 * 128), _f32),
)


def kernel(x, edge_index, W1, b1, W2, b2):
    ei = edge_index.astype(jnp.int32)
    e3 = jnp.pad(ei, ((0, 0), (0, EP - E)),
                 constant_values=NN).reshape(2, TOTC, EB)
    b1v = jnp.tile(b1, 128 // D1).reshape(1, 128)

    degp = _deg_pass(e3)
    degpv = degp.reshape(2, V, 8)
    xwv = _tc_xw(x, W1).reshape(V, 128)
    yv, dinvv = _tc_scale(degpv, xwv)
    sp = _seg_sum(yv.reshape(NP, D1), e3)
    zv = _tc2(sp.reshape(2, V, 128), yv, dinvv, b1v)
    tp = _seg_sum(zv.reshape(NP, D1), e3)
    out = _tc3(tp.reshape(2, V, 128), zv, dinvv, W2, b2.reshape(1, D2))
    return out.reshape(NP, D2)[:NN]


# speedup vs baseline: 1.1607x; 1.0274x over previous
import functools

import jax
import jax.numpy as jnp
from jax import lax
from jax.experimental import pallas as pl
from jax.experimental.pallas import tpu as pltpu
from jax.experimental.pallas import tpu_sc as plsc

NN = 10000
NP = 10240
D0 = 128
D1 = 16
D2 = 32
E = 320000
NW = 32
EB = 320
CH0 = 36
CH1 = 28
TOTC = 16 * (CH0 + CH1)
EP = TOTC * EB
RPS = NP // 16
NBUF = 7
DCH0 = 40
DCH1 = 24


_mesh = plsc.VectorSubcoreMesh(core_axis_name="c", subcore_axis_name="s")
_f32 = jnp.float32
_sc_params = pltpu.CompilerParams(use_tc_tiling_on_sc=False)


@functools.partial(
    pl.kernel,
    out_type=jax.ShapeDtypeStruct((2, NP), _f32),
    mesh=_mesh,
    scratch_types=[
        pltpu.VMEM((DCH0, EB), jnp.int32),
        pltpu.VMEM((EB,), _f32),
        pltpu.VMEM((RPS,), _f32),
        pltpu.VMEM_SHARED((NP,), _f32),
        pltpu.SemaphoreType.DMA,
    ],
    compiler_params=_sc_params,
)
def _deg_pass(e_hbm, out_hbm, dst_v, ones_v, buf_v, acc_sh, sem):
    c = lax.axis_index("c")
    s = lax.axis_index("s")

    @pl.loop(0, EB, step=16)
    def _(i):
        ones_v[pl.ds(i, 16)] = jnp.ones((16,), _f32)

    @pl.loop(0, RPS, step=16)
    def _(i):
        buf_v[pl.ds(i, 16)] = jnp.zeros((16,), _f32)

    pltpu.sync_copy(buf_v, acc_sh.at[pl.ds(s * RPS, RPS)])

    def run(base, ch):
        pltpu.sync_copy(e_hbm.at[1, pl.ds(base, ch)], dst_v.at[pl.ds(0, ch)])
        plsc.subcore_barrier()

        @pl.loop(0, ch, step=4)
        def _(g):
            for b in range(4):
                pltpu.async_copy(ones_v, acc_sh.at[dst_v.at[g + b]], sem,
                                 add=True)
            for b in range(4):
                pltpu.make_async_copy(ones_v, acc_sh.at[dst_v.at[g + b]],
                                      sem).wait()

    @pl.when(c == 0)
    def _():
        run(s * DCH0, DCH0)

    @pl.when(c == 1)
    def _():
        run(16 * DCH0 + s * DCH1, DCH1)

    plsc.subcore_barrier()
    pltpu.sync_copy(acc_sh.at[pl.ds(s * RPS, RPS)], buf_v)
    pltpu.sync_copy(buf_v, out_hbm.at[c, pl.ds(s * RPS, RPS)])


@functools.partial(
    pl.kernel,
    out_type=jax.ShapeDtypeStruct((2, NP, D1), _f32),
    mesh=_mesh,
    scratch_types=[
        pltpu.VMEM((CH0, EB), jnp.int32),
        pltpu.VMEM((CH0, EB), jnp.int32),
        pltpu.VMEM((NBUF, EB, D1), _f32),
        pltpu.VMEM((RPS, D1), _f32),
        pltpu.VMEM_SHARED((NP, D1), _f32),
        pltpu.VMEM_SHARED((NP, D1), _f32),
        pltpu.SemaphoreType.DMA((NBUF,)),
        pltpu.SemaphoreType.DMA((NBUF,)),
    ],
    compiler_params=_sc_params,
)
def _seg_sum(y_hbm, e_hbm, out_hbm, src_v, dst_v, rows_v, buf_v,
             acc_sh, y_sh, gsem, ssem):
    c = lax.axis_index("c")
    s = lax.axis_index("s")

    pltpu.sync_copy(y_hbm.at[pl.ds(s * RPS, RPS)], buf_v)
    pltpu.sync_copy(buf_v, y_sh.at[pl.ds(s * RPS, RPS)])

    @pl.loop(0, RPS)
    def _(i):
        buf_v[i, :] = jnp.zeros((D1,), _f32)

    pltpu.sync_copy(buf_v, acc_sh.at[pl.ds(s * RPS, RPS)])

    def run(base, ch, nbuf):
        pltpu.sync_copy(e_hbm.at[0, pl.ds(base, ch)], src_v.at[pl.ds(0, ch)])
        pltpu.sync_copy(e_hbm.at[1, pl.ds(base, ch)], dst_v.at[pl.ds(0, ch)])
        plsc.subcore_barrier()

        for b in range(nbuf):
            pltpu.async_copy(y_sh.at[src_v.at[b]], rows_v.at[b], gsem.at[b])

        @pl.loop(0, ch, step=nbuf)
        def _(g):
            descs = []
            for b in range(nbuf):
                j = g + b
                pltpu.make_async_copy(
                    y_sh.at[src_v.at[j]], rows_v.at[b], gsem.at[b]).wait()
                descs.append(pltpu.async_copy(
                    rows_v.at[b], acc_sh.at[dst_v.at[j]], ssem.at[b],
                    add=True))
            for b in range(nbuf):
                nj = g + nbuf + b

                @pl.when(nj < ch)
                def _(b=b, nj=nj):
                    descs[b].wait()
                    pltpu.async_copy(y_sh.at[src_v.at[nj]], rows_v.at[b],
                                     gsem.at[b])

        for b in range(nbuf):
            j = ch - nbuf + b
            pltpu.make_async_copy(
                rows_v.at[b], acc_sh.at[dst_v.at[j]], ssem.at[b]).wait()

    @pl.when(c == 0)
    def _():
        run(s * CH0, CH0, 6)

    @pl.when(c == 1)
    def _():
        run(16 * CH0 + s * CH1, CH1, 7)

    plsc.subcore_barrier()
    pltpu.sync_copy(acc_sh.at[pl.ds(s * RPS, RPS)], buf_v)
    pltpu.sync_copy(buf_v, out_hbm.at[c, pl.ds(s * RPS, RPS)])


V = NP * D1 // 128


def _tc_xw_body(x_ref, w1_ref, xw_ref):
    xw = jnp.dot(x_ref[...], w1_ref[...], preferred_element_type=_f32)
    xw_ref[...] = jnp.concatenate([xw, jnp.zeros((NP - NN, D1), _f32)], axis=0)


_tc_xw = pl.pallas_call(
    _tc_xw_body,
    out_shape=jax.ShapeDtypeStruct((NP, D1), _f32),
)


def _tc_scale_body(degp_ref, xw_ref, y_ref, dinv_ref):
    deg8 = degp_ref[0] + degp_ref[1]
    lane = lax.broadcasted_iota(jnp.int32, (8, 128), 1)
    row = lax.broadcasted_iota(jnp.int32, (8, 128), 0)
    rep = (lane // D1 == row).astype(_f32)
    deg = jnp.dot(deg8, rep, preferred_element_type=_f32) + 1.0
    dinv = lax.rsqrt(deg)
    y_ref[...] = xw_ref[...] * dinv
    dinv_ref[...] = dinv


_tc_scale = pl.pallas_call(
    _tc_scale_body,
    out_shape=[jax.ShapeDtypeStruct((V, 128), _f32),
               jax.ShapeDtypeStruct((V, 128), _f32)],
)


def _tc2_body(sp_ref, y_ref, dinv_ref, b1_ref, z_ref):
    agg = dinv_ref[...] * (sp_ref[0] + sp_ref[1] + y_ref[...])
    h = jnp.maximum(agg + b1_ref[...], 0.0)
    z_ref[...] = dinv_ref[...] * h


_tc2 = pl.pallas_call(
    _tc2_body,
    out_shape=jax.ShapeDtypeStruct((V, 128), _f32),
)


def _tc3_body(tp_ref, z_ref, dinv_ref, w2b_ref, b2b_ref, o_ref):
    agg = dinv_ref[...] * (tp_ref[0] + tp_ref[1] + z_ref[...])
    agg = agg[:NN * D1 // 128]
    for i in range(128 // D1):
        blk = jnp.dot(agg[:, D1 * i:D1 * (i + 1)], w2b_ref[...],
                      preferred_element_type=_f32) + b2b_ref[...]
        m = jnp.max(blk, axis=1, keepdims=True)
        lse = jnp.log(jnp.sum(jnp.exp(blk - m), axis=1, keepdims=True)) + m
        o_ref[:, D2 * i:D2 * (i + 1)] = blk - lse


_tc3 = pl.pallas_call(
    _tc3_body,
    out_shape=jax.ShapeDtypeStruct((NN * D1 // 128, 2 * 128), _f32),
)


def kernel(x, edge_index, W1, b1, W2, b2):
    ei = edge_index.astype(jnp.int32)
    e3 = jnp.pad(ei, ((0, 0), (0, EP - E)),
                 constant_values=NN).reshape(2, TOTC, EB)
    b1v = jnp.tile(b1, 128 // D1).reshape(1, 128)

    degp = _deg_pass(e3)
    degpv = degp.reshape(2, V, 8)
    xwv = _tc_xw(x, W1).reshape(V, 128)
    yv, dinvv = _tc_scale(degpv, xwv)
    sp = _seg_sum(yv.reshape(NP, D1), e3)
    zv = _tc2(sp.reshape(2, V, 128), yv, dinvv, b1v)
    tp = _seg_sum(zv.reshape(NP, D1), e3)
    out = _tc3(tp.reshape(2, V, 128), zv, dinvv, W2, b2.reshape(1, D2))
    return out.reshape(NN, D2)
